# baseline (speedup 1.0000x reference)
import jax
import jax.numpy as jnp
from jax import lax
from jax.experimental import pallas as pl
from jax.experimental.pallas import tpu as pltpu
from jax.experimental.pallas import tpu_sc as plsc

M = 1000000
D = 64
B = 16384

NC = 2
NS = 16
NW = NC * NS
R = M // NW
L = 16
MAP = R + (-R) % L


def _body(voxel, idx, pixels, out, idx_v, map_v, rows_l, ids_l, dbuf, sem,
          csem):
    wid = lax.axis_index("s") * NC + lax.axis_index("c")
    lo = wid * R

    flo = pl.multiple_of(wid * (R * D), 128)
    copy = pltpu.async_copy(voxel.at[pl.ds(flo, R * D)],
                            out.at[pl.ds(flo, R * D)], csem)

    pltpu.sync_copy(idx, idx_v)
    lane = lax.iota(jnp.int32, L)

    def p1(c, ptr):
        v = idx_v[pl.ds(c * L, L)]
        m = (v >= lo) & (v < lo + R)
        csum = plsc.cumsum(m.astype(jnp.int32))
        dest = ptr + csum - 1
        plsc.store_scatter(rows_l, [dest], v, mask=m)
        plsc.store_scatter(ids_l, [dest], c * L + lane, mask=m)
        return ptr + jnp.max(csum)

    n = lax.fori_loop(0, B // L, p1, jnp.int32(0))
    nch = (n + L - 1) // L

    def pinit(k, carry):
        m = (k * L + lane) < n
        loc = jnp.where(m, rows_l[pl.ds(k * L, L)] - lo, 0)
        plsc.store_scatter(map_v, [loc], jnp.full((L,), -1, jnp.int32), mask=m)
        return carry

    lax.fori_loop(0, nch, pinit, jnp.int32(0))

    def fix_step(k, carry):
        m = (k * L + lane) < n
        loc = jnp.where(m, rows_l[pl.ds(k * L, L)] - lo, 0)
        idv = ids_l[pl.ds(k * L, L)]
        plsc.store_scatter(map_v, [loc], idv, mask=m)
        for _ in range(L - 1):
            w = plsc.load_gather(map_v, [loc], mask=m)
            upd = m & (idv > w)
            plsc.store_scatter(map_v, [loc], idv, mask=upd)
        return carry

    lax.fori_loop(0, nch, fix_step, jnp.int32(0))

    def p3(k, wptr):
        m = (k * L + lane) < n
        row = rows_l[pl.ds(k * L, L)]
        idv = ids_l[pl.ds(k * L, L)]
        loc = jnp.where(m, row - lo, 0)
        w = plsc.load_gather(map_v, [loc], mask=m)
        win = m & (w == idv)
        csum = plsc.cumsum(win.astype(jnp.int32))
        dest = wptr + csum - 1
        plsc.store_scatter(rows_l, [dest], row, mask=win)
        plsc.store_scatter(ids_l, [dest], idv, mask=win)
        return wptr + jnp.max(csum)

    nwin = lax.fori_loop(0, nch, p3, jnp.int32(0))

    lastp = jnp.full((L,), 0, jnp.int32) + jnp.maximum(nwin - 1, 0)
    lrow = plsc.load_gather(rows_l, [lastp])
    lid = plsc.load_gather(ids_l, [lastp])

    copy.wait()

    def pmove(g, carry):
        m = (g * L + lane) < nwin
        rv = jnp.where(m, rows_l[pl.ds(g * L, L)], lrow)
        iv = jnp.where(m, ids_l[pl.ds(g * L, L)], lid)
        gets = []
        for j in range(L):
            src = pixels.at[pl.ds(pl.multiple_of(iv[j] * D, D), D)]
            gets.append(pltpu.async_copy(src, dbuf.at[pl.ds(j * D, D)], sem))
        for h in gets:
            h.wait()
        puts = []
        for j in range(L):
            dst = out.at[pl.ds(pl.multiple_of(rv[j] * D, D), D)]
            puts.append(pltpu.async_copy(dbuf.at[pl.ds(j * D, D)], dst, sem))
        for h in puts:
            h.wait()
        return carry

    lax.fori_loop(0, (nwin + L - 1) // L, pmove, jnp.int32(0))


_scatter = pl.kernel(
    _body,
    out_type=jax.ShapeDtypeStruct((M * D,), jnp.float32),
    mesh=plsc.VectorSubcoreMesh(core_axis_name="c", subcore_axis_name="s"),
    compiler_params=pltpu.CompilerParams(needs_layout_passes=False),
    scratch_types=[
        pltpu.VMEM((B,), jnp.int32),
        pltpu.VMEM((MAP,), jnp.int32),
        pltpu.VMEM((B,), jnp.int32),
        pltpu.VMEM((B,), jnp.int32),
        pltpu.VMEM((L * D,), jnp.float32),
        pltpu.SemaphoreType.DMA,
        pltpu.SemaphoreType.DMA,
    ],
)


@jax.jit
def kernel(voxel, scatter_indices, pixels):
    flat = _scatter(voxel.reshape(M * D), scatter_indices.reshape(B),
                    pixels.reshape(B * D))
    return flat.reshape(M, D)


# speedup vs baseline: 6.1405x; 6.1405x over previous
import jax
import jax.numpy as jnp
from jax import lax
from jax.experimental import pallas as pl
from jax.experimental.pallas import tpu as pltpu
from jax.experimental.pallas import tpu_sc as plsc

M = 1000000
D = 64
B = 16384

NC = 2
NS = 16
NW = NC * NS
R = M // NW
L = 16
MAP = R + (-R) % L
CPW = 20000


def _body(voxel, idx, pixels, out, idx_v, map_v, rows_l, ids_l, dbuf, cbuf,
          sem, csem, wsem):
    wid = lax.axis_index("s") * NC + lax.axis_index("c")
    lo = wid * R

    flo = pl.multiple_of(wid * (R * D), 128)

    pltpu.sync_copy(idx, idx_v)
    lane = lax.iota(jnp.int32, L)

    def p1(c, ptr):
        v = idx_v[pl.ds(c * L, L)]
        m = (v >= lo) & (v < lo + R)
        csum = plsc.cumsum(m.astype(jnp.int32))
        dest = ptr + csum - 1
        plsc.store_scatter(rows_l, [dest], v, mask=m)
        plsc.store_scatter(ids_l, [dest], c * L + lane, mask=m)
        return ptr + jnp.max(csum)

    n = lax.fori_loop(0, B // L, p1, jnp.int32(0))
    nch = (n + L - 1) // L

    def pinit(k, carry):
        m = (k * L + lane) < n
        loc = jnp.where(m, rows_l[pl.ds(k * L, L)] - lo, 0)
        plsc.store_scatter(map_v, [loc], jnp.full((L,), -1, jnp.int32), mask=m)
        return carry

    lax.fori_loop(0, nch, pinit, jnp.int32(0))

    def fix_step(k, carry):
        m = (k * L + lane) < n
        loc = jnp.where(m, rows_l[pl.ds(k * L, L)] - lo, 0)
        idv = ids_l[pl.ds(k * L, L)]
        plsc.store_scatter(map_v, [loc], idv, mask=m)
        for _ in range(L - 1):
            w = plsc.load_gather(map_v, [loc], mask=m)
            upd = m & (idv > w)
            plsc.store_scatter(map_v, [loc], idv, mask=upd)
        return carry

    lax.fori_loop(0, nch, fix_step, jnp.int32(0))

    def p3(k, wptr):
        m = (k * L + lane) < n
        row = rows_l[pl.ds(k * L, L)]
        idv = ids_l[pl.ds(k * L, L)]
        loc = jnp.where(m, row - lo, 0)
        w = plsc.load_gather(map_v, [loc], mask=m)
        win = m & (w == idv)
        csum = plsc.cumsum(win.astype(jnp.int32))
        dest = wptr + csum - 1
        plsc.store_scatter(rows_l, [dest], row, mask=win)
        plsc.store_scatter(ids_l, [dest], idv, mask=win)
        return wptr + jnp.max(csum)

    nwin = lax.fori_loop(0, nch, p3, jnp.int32(0))

    lastp = jnp.full((L,), 0, jnp.int32) + jnp.maximum(nwin - 1, 0)
    lrow = plsc.load_gather(rows_l, [lastp])
    lid = plsc.load_gather(ids_l, [lastp])

    nckv = (R * D) // CPW
    pltpu.async_copy(voxel.at[pl.ds(flo, CPW)], cbuf.at[pl.ds(0, CPW)], csem)

    def cstep(t, carry):
        par = (t % 2) * CPW
        nxt = ((t + 1) % 2) * CPW
        pltpu.make_async_copy(voxel.at[pl.ds(flo, CPW)],
                              cbuf.at[pl.ds(par, CPW)], csem).wait()

        @pl.when(t + 1 < nckv)
        def _pref():
            src = voxel.at[pl.ds(flo + (t + 1) * CPW, CPW)]
            pltpu.async_copy(src, cbuf.at[pl.ds(nxt, CPW)], csem)

        @pl.when(t > 0)
        def _dr():
            pltpu.make_async_copy(cbuf.at[pl.ds(nxt, CPW)],
                                  out.at[pl.ds(flo, CPW)], wsem).wait()

        pltpu.async_copy(cbuf.at[pl.ds(par, CPW)],
                         out.at[pl.ds(flo + t * CPW, CPW)], wsem)
        return carry

    lax.fori_loop(0, nckv, cstep, jnp.int32(0))
    pltpu.make_async_copy(cbuf.at[pl.ds(0, CPW)],
                          out.at[pl.ds(flo, CPW)], wsem).wait()

    def pmove(g, carry):
        m = (g * L + lane) < nwin
        rv = jnp.where(m, rows_l[pl.ds(g * L, L)], lrow)
        iv = jnp.where(m, ids_l[pl.ds(g * L, L)], lid)
        gets = []
        for j in range(L):
            src = pixels.at[pl.ds(pl.multiple_of(iv[j] * D, D), D)]
            gets.append(pltpu.async_copy(src, dbuf.at[pl.ds(j * D, D)], sem))
        for h in gets:
            h.wait()
        puts = []
        for j in range(L):
            dst = out.at[pl.ds(pl.multiple_of(rv[j] * D, D), D)]
            puts.append(pltpu.async_copy(dbuf.at[pl.ds(j * D, D)], dst, sem))
        for h in puts:
            h.wait()
        return carry

    lax.fori_loop(0, (nwin + L - 1) // L, pmove, jnp.int32(0))


_scatter = pl.kernel(
    _body,
    out_type=jax.ShapeDtypeStruct((M * D,), jnp.float32),
    mesh=plsc.VectorSubcoreMesh(core_axis_name="c", subcore_axis_name="s"),
    compiler_params=pltpu.CompilerParams(needs_layout_passes=False),
    scratch_types=[
        pltpu.VMEM((B,), jnp.int32),
        pltpu.VMEM((MAP,), jnp.int32),
        pltpu.VMEM((B,), jnp.int32),
        pltpu.VMEM((B,), jnp.int32),
        pltpu.VMEM((L * D,), jnp.float32),
        pltpu.VMEM((2 * CPW,), jnp.float32),
        pltpu.SemaphoreType.DMA,
        pltpu.SemaphoreType.DMA,
        pltpu.SemaphoreType.DMA,
    ],
)


@jax.jit
def kernel(voxel, scatter_indices, pixels):
    flat = _scatter(voxel.reshape(M * D), scatter_indices.reshape(B),
                    pixels.reshape(B * D))
    return flat.reshape(M, D)


# speedup vs baseline: 7.3258x; 1.1930x over previous
import jax
import jax.numpy as jnp
from jax import lax
from jax.experimental import pallas as pl
from jax.experimental.pallas import tpu as pltpu
from jax.experimental.pallas import tpu_sc as plsc

M = 1000000
D = 64
B = 16384

NC = 2
NS = 16
NW = NC * NS
R = 31248
TAIL = M - NW * R
L = 16
MAP = R + TAIL
CR = 144
NCH = R // CR
CWCAP = CR + L


def _body(voxel, idx, pixels, out, idx_v, map_v, pk_l, cw_l, cbuf,
          csem, wsem, psem):
    wid = lax.axis_index("s") * NC + lax.axis_index("c")
    last = wid == NW - 1
    lo = pl.multiple_of(wid * R, 8)
    hi = jnp.where(last, M, lo + R)

    pltpu.sync_copy(idx, idx_v)
    lane = lax.iota(jnp.int32, L)

    def p1(c, ptr):
        v = idx_v[pl.ds(c * L, L)]
        m = (v >= lo) & (v < hi)
        pk = jnp.where(m, v - lo, 0) * B + (c * L + lane)
        csum = plsc.cumsum(m.astype(jnp.int32))
        plsc.store_scatter(pk_l, [ptr + csum - 1], pk, mask=m)
        return ptr + csum[L - 1]

    n = lax.fori_loop(0, B // L, p1, jnp.int32(0))
    nch = (n + L - 1) // L

    def pinit(k, carry):
        m = (k * L + lane) < n
        loc = jnp.where(m, lax.shift_right_logical(pk_l[pl.ds(k * L, L)], 14),
                        0)
        plsc.store_scatter(map_v, [loc], jnp.full((L,), -1, jnp.int32), mask=m)
        return carry

    lax.fori_loop(0, nch, pinit, jnp.int32(0))

    def fix_step(k, carry):
        m = (k * L + lane) < n
        pk = pk_l[pl.ds(k * L, L)]
        loc = jnp.where(m, lax.shift_right_logical(pk, 14), 0)
        plsc.store_scatter(map_v, [loc], pk, mask=m)
        for _ in range(L - 1):
            w = plsc.load_gather(map_v, [loc], mask=m)
            upd = m & (pk > w)
            plsc.store_scatter(map_v, [loc], pk, mask=upd)
        return carry

    lax.fori_loop(0, nch, fix_step, jnp.int32(0))

    def p3(k, wptr):
        m = (k * L + lane) < n
        pk = pk_l[pl.ds(k * L, L)]
        loc = jnp.where(m, lax.shift_right_logical(pk, 14), 0)
        w = plsc.load_gather(map_v, [loc], mask=m)
        win = m & (w == pk)
        csum = plsc.cumsum(win.astype(jnp.int32))
        plsc.store_scatter(pk_l, [wptr + csum - 1], pk, mask=win)
        return wptr + csum[L - 1]

    nwin = lax.fori_loop(0, nch, p3, jnp.int32(0))
    nwch = (nwin + L - 1) // L

    def merge_into(off, cstart, crows):
        def scan(k, cnt):
            mw = (k * L + lane) < nwin
            pk = pk_l[pl.ds(k * L, L)]
            loc = lax.shift_right_logical(pk, 14)
            inb = mw & (loc >= cstart) & (loc < cstart + crows)
            csum = plsc.cumsum(inb.astype(jnp.int32))
            plsc.store_scatter(cw_l, [cnt + csum - 1], pk, mask=inb)
            return cnt + csum[L - 1]

        cnt = lax.fori_loop(0, nwch, scan, jnp.int32(0))

        def apply16(b, carry):
            mv = (b * L + lane) < cnt
            pk = cw_l[pl.ds(b * L, L)]
            pk = jnp.where(mv, pk, jnp.zeros((L,), jnp.int32) + pk[0])
            rv = lax.shift_right_logical(pk, 14) - cstart
            iv = pk & (B - 1)
            gets = []
            for j in range(L):
                src = pixels.at[pl.ds(pl.multiple_of(iv[j] * D, 8), D)]
                gets.append(
                    pltpu.async_copy(src, cbuf.at[off + rv[j]], psem))
            for h in gets:
                h.wait()
            return carry

        lax.fori_loop(0, (cnt + L - 1) // L, apply16, jnp.int32(0))

    pltpu.async_copy(voxel.at[pl.ds(lo, CR)], cbuf.at[pl.ds(0, CR)], csem)

    def cstep(t, carry):
        par = (t % 2) * CR
        nxt = ((t + 1) % 2) * CR
        base = pl.multiple_of(lo + t * CR, 8)
        pltpu.make_async_copy(voxel.at[pl.ds(base, CR)],
                              cbuf.at[pl.ds(par, CR)], csem).wait()

        @pl.when(t + 1 < NCH)
        def _pref():
            src = voxel.at[pl.ds(pl.multiple_of(base + CR, 8), CR)]
            pltpu.async_copy(src, cbuf.at[pl.ds(nxt, CR)], csem)

        merge_into(par, t * CR, CR)

        @pl.when(t > 0)
        def _dr():
            pltpu.make_async_copy(cbuf.at[pl.ds(nxt, CR)],
                                  out.at[pl.ds(base, CR)], wsem).wait()

        pltpu.async_copy(cbuf.at[pl.ds(par, CR)], out.at[pl.ds(base, CR)],
                         wsem)
        return carry

    lax.fori_loop(0, NCH, cstep, jnp.int32(0))
    pltpu.make_async_copy(cbuf.at[pl.ds(0, CR)], out.at[pl.ds(0, CR)],
                          wsem).wait()

    @pl.when(last)
    def _tail():
        pltpu.sync_copy(voxel.at[pl.ds(M - TAIL, TAIL)],
                        cbuf.at[pl.ds(0, TAIL)])
        merge_into(jnp.int32(0), jnp.int32(R), TAIL)
        pltpu.sync_copy(cbuf.at[pl.ds(0, TAIL)], out.at[pl.ds(M - TAIL, TAIL)])


_scatter = pl.kernel(
    _body,
    out_type=jax.ShapeDtypeStruct((M, D), jnp.float32),
    mesh=plsc.VectorSubcoreMesh(core_axis_name="c", subcore_axis_name="s"),
    compiler_params=pltpu.CompilerParams(needs_layout_passes=False),
    scratch_types=[
        pltpu.VMEM((B,), jnp.int32),
        pltpu.VMEM((MAP,), jnp.int32),
        pltpu.VMEM((B,), jnp.int32),
        pltpu.VMEM((CWCAP,), jnp.int32),
        pltpu.VMEM((2 * CR, D), jnp.float32),
        pltpu.SemaphoreType.DMA,
        pltpu.SemaphoreType.DMA,
        pltpu.SemaphoreType.DMA,
    ],
)


@jax.jit
def kernel(voxel, scatter_indices, pixels):
    return _scatter(voxel, scatter_indices.reshape(B), pixels.reshape(B * D))
